# baseline (speedup 1.0000x reference)
import functools

import jax
import jax.numpy as jnp
from jax import lax
from jax.experimental import pallas as pl
from jax.experimental.pallas import tpu as pltpu
from jax.experimental.pallas import tpu_sc as plsc


@functools.lru_cache(maxsize=None)
def _make_gather(V, D, B):
    info = plsc.get_sparse_core_info()
    NC, NS = info.num_cores, info.num_subcores
    NW = NC * NS
    assert D % info.num_lanes == 0 and B % (8 * NW) == 0
    b_per_w = B // NW
    mesh = plsc.VectorSubcoreMesh(core_axis_name="c", subcore_axis_name="s")

    n_chunks = 16
    n_win = 4
    C = b_per_w // n_chunks

    stage_rows = 64
    n_full = V // stage_rows
    rem = V - n_full * stage_rows

    @functools.partial(
        pl.kernel, mesh=mesh,
        out_type=jax.ShapeDtypeStruct((B, D), jnp.float32),
        scratch_types=[
            pltpu.VMEM((b_per_w,), jnp.int32),
            pltpu.VMEM((n_chunks, C, D), jnp.float32),
            pltpu.VMEM_SHARED((V, D), jnp.float32),
            pltpu.SemaphoreType.DMA((n_win,)),
            pltpu.SemaphoreType.DMA((n_win,)),
            pltpu.SemaphoreType.DMA,
        ],
    )
    def k(table_hbm, idx_hbm, out_hbm, idx_v, rows_v, table_sp, gsem, osem,
          isem):
        cid = lax.axis_index("c")
        sid = lax.axis_index("s")
        wid = sid * NC + cid
        base = wid * b_per_w
        idx_cp = pltpu.async_copy(idx_hbm.at[pl.ds(base, b_per_w)], idx_v, isem)
        for s in range(n_full):
            @pl.when(sid == s)
            def _():
                pltpu.sync_copy(
                    table_hbm.at[pl.ds(s * stage_rows, stage_rows)],
                    table_sp.at[pl.ds(s * stage_rows, stage_rows)])
        if rem:
            @pl.when(sid == n_full % NS)
            def _():
                pltpu.sync_copy(
                    table_hbm.at[pl.ds(n_full * stage_rows, rem)],
                    table_sp.at[pl.ds(n_full * stage_rows, rem)])
        plsc.subcore_barrier()
        idx_cp.wait()
        def gather(c):
            return pltpu.async_copy(
                table_sp.at[idx_v.at[pl.ds(c * C, C)]], rows_v.at[c],
                gsem.at[c % n_win])

        gathers = [gather(c) for c in range(n_win)]
        gathers += [None] * (n_chunks - n_win)
        outs = [None] * n_chunks
        for c in range(n_chunks):
            gathers[c].wait()
            if c >= n_win:
                outs[c - n_win].wait()
            outs[c] = pltpu.async_copy(
                rows_v.at[c], out_hbm.at[pl.ds(base + c * C, C)],
                osem.at[c % n_win])
            if c + n_win < n_chunks:
                gathers[c + n_win] = gather(c + n_win)
        for c in range(n_chunks - n_win, n_chunks):
            outs[c].wait()

    return k


def kernel(x, t, embeddings):
    V, D = embeddings.shape
    B = t.shape[0]
    out = _make_gather(V, D, B)(embeddings, t.astype(jnp.int32))
    return out[:, :, None, None]


# speedup vs baseline: 1.0022x; 1.0022x over previous
import functools

import jax
import jax.numpy as jnp
from jax import lax
from jax.experimental import pallas as pl
from jax.experimental.pallas import tpu as pltpu
from jax.experimental.pallas import tpu_sc as plsc


@functools.lru_cache(maxsize=None)
def _make_gather(V, D, B):
    info = plsc.get_sparse_core_info()
    NC, NS = info.num_cores, info.num_subcores
    NW = NC * NS
    assert D % info.num_lanes == 0 and B % (8 * NW) == 0
    b_per_w = B // NW
    mesh = plsc.VectorSubcoreMesh(core_axis_name="c", subcore_axis_name="s")

    n_chunks = 8
    C = b_per_w // n_chunks

    stage_rows = 64
    n_full = V // stage_rows
    rem = V - n_full * stage_rows

    @functools.partial(
        pl.kernel, mesh=mesh,
        out_type=jax.ShapeDtypeStruct((B, D), jnp.float32),
        scratch_types=[
            pltpu.VMEM((b_per_w,), jnp.int32),
            pltpu.VMEM((n_chunks, C, D), jnp.float32),
            pltpu.VMEM_SHARED((V, D), jnp.float32),
            pltpu.SemaphoreType.DMA((n_chunks,)),
            pltpu.SemaphoreType.DMA((n_chunks,)),
            pltpu.SemaphoreType.DMA,
        ],
    )
    def k(table_hbm, idx_hbm, out_hbm, idx_v, rows_v, table_sp, gsem, osem,
          isem):
        cid = lax.axis_index("c")
        sid = lax.axis_index("s")
        wid = sid * NC + cid
        base = wid * b_per_w
        idx_cp = pltpu.async_copy(idx_hbm.at[pl.ds(base, b_per_w)], idx_v, isem)
        for s in range(n_full):
            @pl.when(sid == s)
            def _():
                pltpu.sync_copy(
                    table_hbm.at[pl.ds(s * stage_rows, stage_rows)],
                    table_sp.at[pl.ds(s * stage_rows, stage_rows)])
        if rem:
            @pl.when(sid == n_full % NS)
            def _():
                pltpu.sync_copy(
                    table_hbm.at[pl.ds(n_full * stage_rows, rem)],
                    table_sp.at[pl.ds(n_full * stage_rows, rem)])
        plsc.subcore_barrier()
        idx_cp.wait()
        gathers = [
            pltpu.async_copy(
                table_sp.at[idx_v.at[pl.ds(c * C, C)]], rows_v.at[c], gsem.at[c])
            for c in range(n_chunks)
        ]
        outs = []
        for c in range(n_chunks):
            gathers[c].wait()
            outs.append(pltpu.async_copy(
                rows_v.at[c], out_hbm.at[pl.ds(base + c * C, C)], osem.at[c]))
        for o in outs:
            o.wait()

    return k


def kernel(x, t, embeddings):
    V, D = embeddings.shape
    B = t.shape[0]
    out = _make_gather(V, D, B)(embeddings, t.astype(jnp.int32))
    return out[:, :, None, None]
